# Initial kernel scaffold; baseline (speedup 1.0000x reference)
import jax
import jax.numpy as jnp
from jax.experimental import pallas as pl


def kernel(embedding_matrix, motifs, label):
    raise NotImplementedError("write your pallas kernel here")


# speedup vs baseline: 2.5243x; 2.5243x over previous
import functools

import jax
import jax.numpy as jnp
from jax import lax
from jax.experimental import pallas as pl
from jax.experimental.pallas import tpu as pltpu
from jax.experimental.pallas import tpu_sc as plsc

NC = 2
NS = 16
NW = NC * NS
L = 16

N_NODES = 100000
D = 16
B = 16384
MOTIF = 3

BPW = B // NW
IPW = BPW * MOTIF
CHUNK = 128
NCHUNK = IPW // CHUNK
GROUPS = BPW // L


def _sc_body(motifs_hbm, label_hbm, table_hbm, reward_hbm, parts_hbm,
             idx_v, rows_v, label_v, reward_v, tmat_v, parts_v, sem):
    wid = lax.axis_index("s") * NC + lax.axis_index("c")

    pltpu.sync_copy(motifs_hbm.at[wid], idx_v)
    pltpu.sync_copy(label_hbm.at[wid], label_v)

    copies = []
    for j in range(NCHUNK):
        copies.append(pltpu.async_copy(
            table_hbm.at[idx_v.at[j]],
            rows_v.at[pl.ds(j * CHUNK, CHUNK)],
            sem))
    for c in copies:
        c.wait()

    lane = lax.iota(jnp.int32, L)
    tr_base = lane * L

    def group(g, acc):
        e0 = g * L
        for j in range(L):
            r = MOTIF * (e0 + j)
            prod = rows_v[r] * rows_v[r + 1] * rows_v[r + 2]
            plsc.store_scatter(tmat_v, [tr_base + j], prod)
        score = tmat_v[pl.ds(0, L)]
        for d in range(1, D):
            score = score + tmat_v[pl.ds(d * L, L)]
        p = jnp.clip(1.0 - jnp.exp(-score), 1e-05, 1.0)
        reward_v[pl.ds(e0, L)] = 1.0 - p
        lbl = label_v[pl.ds(e0, L)]
        return acc + (lbl * p + (1.0 - lbl) * (1.0 - p))

    acc = lax.fori_loop(0, GROUPS, group, jnp.zeros((L,), jnp.float32))

    parts_v[...] = acc
    pltpu.sync_copy(reward_v, reward_hbm.at[wid])
    pltpu.sync_copy(parts_v, parts_hbm.at[wid])


@functools.partial(
    pl.kernel,
    out_type=[jax.ShapeDtypeStruct((NW, BPW), jnp.float32),
              jax.ShapeDtypeStruct((NW, L), jnp.float32)],
    mesh=plsc.VectorSubcoreMesh(core_axis_name="c", subcore_axis_name="s"),
    compiler_params=pltpu.CompilerParams(needs_layout_passes=False,
                                         use_tc_tiling_on_sc=False),
    scratch_types=[
        pltpu.VMEM((NCHUNK, CHUNK), jnp.int32),
        pltpu.VMEM((IPW, D), jnp.float32),
        pltpu.VMEM((BPW,), jnp.float32),
        pltpu.VMEM((BPW,), jnp.float32),
        pltpu.VMEM((D * L,), jnp.float32),
        pltpu.VMEM((L,), jnp.float32),
        pltpu.SemaphoreType.DMA,
    ],
)
def _sc_kernel(motifs_hbm, label_hbm, table_hbm, reward_hbm, parts_hbm,
               idx_v, rows_v, label_v, reward_v, tmat_v, parts_v, sem):
    _sc_body(motifs_hbm, label_hbm, table_hbm, reward_hbm, parts_hbm,
             idx_v, rows_v, label_v, reward_v, tmat_v, parts_v, sem)


def _loss_body(parts_ref, out_ref):
    out_ref[0, 0] = -jnp.sum(parts_ref[...])


def _loss_finish(parts):
    return pl.pallas_call(
        _loss_body,
        out_shape=jax.ShapeDtypeStruct((1, 1), jnp.float32),
        out_specs=pl.BlockSpec(memory_space=pltpu.SMEM),
    )(parts)


@jax.jit
def kernel(embedding_matrix, motifs, label):
    motifs_w = motifs.astype(jnp.int32).reshape(NW, NCHUNK, CHUNK)
    label_w = label.reshape(NW, BPW)
    reward_w, parts = _sc_kernel(motifs_w, label_w, embedding_matrix)
    loss = _loss_finish(parts)[0, 0]
    return (loss, reward_w.reshape(B))


# speedup vs baseline: 3.1184x; 1.2354x over previous
import functools

import jax
import jax.numpy as jnp
from jax import lax
from jax.experimental import pallas as pl
from jax.experimental.pallas import tpu as pltpu
from jax.experimental.pallas import tpu_sc as plsc

NC = 2
NS = 16
NW = NC * NS
L = 16

N_NODES = 100000
D = 16
B = 16384
MOTIF = 3

BPW = B // NW
CHUNK = 128
NCH = BPW // CHUNK
GROUPS = BPW // L

DET_C = 4096
DET_G = -(-N_NODES // DET_C)


def _detile_body(x_ref, o_ref):
    x3 = x_ref[...].T.reshape(DET_C // 8, 8, D)
    o_ref[...] = jnp.concatenate([x3[:, j, :] for j in range(8)], axis=1)


def _detile(emb_t):
    return pl.pallas_call(
        _detile_body,
        grid=(DET_G,),
        in_specs=[pl.BlockSpec((D, DET_C), lambda k: (0, k))],
        out_specs=pl.BlockSpec((DET_C * D // 128, 128), lambda k: (k, 0)),
        out_shape=jax.ShapeDtypeStruct((N_NODES * D // 128, 128), jnp.float32),
    )(emb_t)


def _sc_body(motifs_hbm, label_hbm, table_hbm, reward_hbm, parts_hbm,
             idx_v, rows_v, label_v, reward_v, tmat_v, parts_v, sem):
    wid = lax.axis_index("s") * NC + lax.axis_index("c")

    pltpu.sync_copy(motifs_hbm.at[:, wid], idx_v)
    pltpu.sync_copy(label_hbm.at[wid], label_v)

    copies = []
    for m in range(MOTIF):
        for j in range(NCH):
            copies.append(pltpu.async_copy(
                table_hbm.at[idx_v.at[m, j]],
                rows_v.at[pl.ds((m * NCH + j) * CHUNK, CHUNK)],
                sem))
    for c in copies:
        c.wait()

    lane = lax.iota(jnp.int32, L)
    tr_base = lane * L

    def group(g, acc):
        e0 = g * L
        for j in range(L):
            e = e0 + j
            prod = rows_v[e] * rows_v[BPW + e] * rows_v[2 * BPW + e]
            plsc.store_scatter(tmat_v, [tr_base + j], prod)
        score = tmat_v[pl.ds(0, L)]
        for d in range(1, D):
            score = score + tmat_v[pl.ds(d * L, L)]
        p = jnp.clip(1.0 - jnp.exp(-score), 1e-05, 1.0)
        reward_v[pl.ds(e0, L)] = 1.0 - p
        lbl = label_v[pl.ds(e0, L)]
        return acc + (lbl * p + (1.0 - lbl) * (1.0 - p))

    acc = lax.fori_loop(0, GROUPS, group, jnp.zeros((L,), jnp.float32))

    parts_v[...] = acc
    pltpu.sync_copy(reward_v, reward_hbm.at[wid])
    pltpu.sync_copy(parts_v, parts_hbm.at[wid])


@functools.partial(
    pl.kernel,
    out_type=[jax.ShapeDtypeStruct((NW, BPW), jnp.float32),
              jax.ShapeDtypeStruct((NW, L), jnp.float32)],
    mesh=plsc.VectorSubcoreMesh(core_axis_name="c", subcore_axis_name="s"),
    compiler_params=pltpu.CompilerParams(needs_layout_passes=False,
                                         use_tc_tiling_on_sc=False),
    scratch_types=[
        pltpu.VMEM((MOTIF, NCH, CHUNK), jnp.int32),
        pltpu.VMEM((MOTIF * BPW, D), jnp.float32),
        pltpu.VMEM((BPW,), jnp.float32),
        pltpu.VMEM((BPW,), jnp.float32),
        pltpu.VMEM((D * L,), jnp.float32),
        pltpu.VMEM((L,), jnp.float32),
        pltpu.SemaphoreType.DMA,
    ],
)
def _sc_kernel(motifs_hbm, label_hbm, table_hbm, reward_hbm, parts_hbm,
               idx_v, rows_v, label_v, reward_v, tmat_v, parts_v, sem):
    _sc_body(motifs_hbm, label_hbm, table_hbm, reward_hbm, parts_hbm,
             idx_v, rows_v, label_v, reward_v, tmat_v, parts_v, sem)


def _loss_body(parts_ref, out_ref):
    out_ref[0, 0] = -jnp.sum(parts_ref[...])


def _loss_finish(parts):
    return pl.pallas_call(
        _loss_body,
        out_shape=jax.ShapeDtypeStruct((1, 1), jnp.float32),
        out_specs=pl.BlockSpec(memory_space=pltpu.SMEM),
    )(parts)


@jax.jit
def kernel(embedding_matrix, motifs, label):
    table = _detile(embedding_matrix.T).reshape(N_NODES, D)
    motifs_w = motifs.T.astype(jnp.int32).reshape(MOTIF, NW, NCH, CHUNK)
    label_w = label.reshape(NW, BPW)
    reward_w, parts = _sc_kernel(motifs_w, label_w, table)
    loss = _loss_finish(parts)[0, 0]
    return (loss, reward_w.reshape(B))
